# Initial kernel scaffold; baseline (speedup 1.0000x reference)
import jax
import jax.numpy as jnp
from jax.experimental import pallas as pl


def kernel(x, batch, w1a, b1a, g1a, bt1a, w1b, b1b, w2a, b2a, g2a, bt2a, w2b, b2b, w3a, b3a, g3a, bt3a, w3b, b3b, w4, b4, w5, b5, w6, b6, w7, b7):
    raise NotImplementedError("write your pallas kernel here")


# speedup vs baseline: 7.6816x; 7.6816x over previous
import functools

import jax
import jax.numpy as jnp
from jax import lax
from jax.experimental import pallas as pl
from jax.experimental.pallas import tpu as pltpu
from jax.experimental.pallas import tpu_sc as plsc

_N = 4096
_K = 20
_F32 = jnp.float32


def _linear_pq(feat, wi, wj, ba):
    n, d = feat.shape
    C = wi.shape[-1]
    Cq = wj.shape[-1]
    blk = 512

    def body(f_ref, wi_ref, wj_ref, ba_ref, p_ref, q_ref):
        f = f_ref[...]
        if d == 1:
            p_ref[...] = f * wi_ref[...] + ba_ref[...]
            q_ref[...] = f * wj_ref[...]
        else:
            p_ref[...] = (
                jnp.dot(f, wi_ref[...], preferred_element_type=_F32) + ba_ref[...]
            )
            q_ref[...] = jnp.dot(f, wj_ref[...], preferred_element_type=_F32)

    return pl.pallas_call(
        body,
        grid=(n // blk,),
        in_specs=[
            pl.BlockSpec((blk, d), lambda i: (i, 0)),
            pl.BlockSpec((wi.shape[0], C), lambda i: (0, 0)),
            pl.BlockSpec((wj.shape[0], Cq), lambda i: (0, 0)),
            pl.BlockSpec((1, C), lambda i: (0, 0)),
        ],
        out_specs=[
            pl.BlockSpec((blk, C), lambda i: (i, 0)),
            pl.BlockSpec((blk, Cq), lambda i: (i, 0)),
        ],
        out_shape=[
            jax.ShapeDtypeStruct((n, C), _F32),
            jax.ShapeDtypeStruct((n, Cq), _F32),
        ],
    )(feat, wi, wj, ba)


def _dist_topk(feat, featT, batch_col, batch_row):
    n, d = feat.shape
    blk = 128

    def body(f_ref, ft_ref, b_ref, bt_ref, idx_ref):
        fb = f_ref[...]
        ft = ft_ref[...]
        d2r = jnp.sum(ft * ft, axis=0, keepdims=True)
        d2b = jnp.sum(fb * fb, axis=1, keepdims=True)
        if d == 1:
            cross = fb * ft
        else:
            cross = jnp.dot(fb, ft, preferred_element_type=_F32)
        dist = d2b + d2r - 2.0 * cross
        dist = jnp.where(b_ref[...] != bt_ref[...], 1e9, dist)
        cols = lax.broadcasted_iota(jnp.int32, (blk, n), 1)
        picks = []
        for _ in range(_K):
            m = jnp.min(dist, axis=1, keepdims=True)
            sel = jnp.where(dist == m, cols, jnp.int32(2**30))
            j = jnp.min(sel, axis=1, keepdims=True)
            picks.append(j)
            dist = jnp.where(cols == j, jnp.float32(jnp.inf), dist)
        idx_ref[...] = jnp.concatenate(picks, axis=1)

    return pl.pallas_call(
        body,
        grid=(n // blk,),
        in_specs=[
            pl.BlockSpec((blk, d), lambda i: (i, 0)),
            pl.BlockSpec((d, n), lambda i: (0, 0)),
            pl.BlockSpec((blk, 1), lambda i: (i, 0)),
            pl.BlockSpec((1, n), lambda i: (0, 0)),
        ],
        out_specs=pl.BlockSpec((blk, _K), lambda i: (i, 0)),
        out_shape=jax.ShapeDtypeStruct((n, _K), jnp.int32),
    )(feat, featT, batch_col, batch_row)


def _sc_gather(Q, idx_flat):
    B = idx_flat.shape[0]
    C = Q.shape[1]
    NW = 32
    CH = 128
    per_w = B // NW
    nch = per_w // CH
    mesh = plsc.VectorSubcoreMesh(core_axis_name="c", subcore_axis_name="s")

    @functools.partial(
        pl.kernel,
        mesh=mesh,
        out_type=jax.ShapeDtypeStruct((B, C), _F32),
        scratch_types=[
            pltpu.VMEM((CH,), jnp.int32),
            pltpu.VMEM((CH,), jnp.int32),
            pltpu.VMEM((CH, C), _F32),
            pltpu.VMEM((CH, C), _F32),
            pltpu.SemaphoreType.DMA,
            pltpu.SemaphoreType.DMA,
        ],
    )
    def gk(q_hbm, idx_hbm, out_hbm, idx_a, idx_b, rows_a, rows_b, sem_a, sem_b):
        wid = lax.axis_index("s") * 2 + lax.axis_index("c")
        base = wid * per_w
        idx_v = (idx_a, idx_b)
        rows_v = (rows_a, rows_b)
        sems = (sem_a, sem_b)
        pltpu.sync_copy(idx_hbm.at[pl.ds(base, CH)], idx_a)
        gat = pltpu.async_copy(q_hbm.at[idx_a], rows_a, sem_a)
        for c in range(nch):
            s = c % 2
            o = 1 - s
            if c + 1 < nch:
                off_n = base + (c + 1) * CH
                pltpu.sync_copy(idx_hbm.at[pl.ds(off_n, CH)], idx_v[o])
                nxt = pltpu.async_copy(q_hbm.at[idx_v[o]], rows_v[o], sems[o])
            gat.wait()
            pltpu.sync_copy(rows_v[s], out_hbm.at[pl.ds(base + c * CH, CH)])
            if c + 1 < nch:
                gat = nxt

    return gk(Q, idx_flat)


def _edge_stats(P, QgR, Cp):
    n, C = P.shape
    blk = 128
    nb = n // blk

    def body(p_ref, qg_ref, s1_ref, s2_ref):
        p = p_ref[...]
        s1 = jnp.zeros((1, C), _F32)
        s2 = jnp.zeros((1, C), _F32)
        for k in range(_K):
            z = p + qg_ref[:, k * Cp : k * Cp + C]
            s1 = s1 + jnp.sum(z, axis=0, keepdims=True)
            s2 = s2 + jnp.sum(z * z, axis=0, keepdims=True)
        s1_ref[...] = s1.reshape(1, 1, C)
        s2_ref[...] = s2.reshape(1, 1, C)

    return pl.pallas_call(
        body,
        grid=(nb,),
        in_specs=[
            pl.BlockSpec((blk, C), lambda i: (i, 0)),
            pl.BlockSpec((blk, _K * Cp), lambda i: (i, 0)),
        ],
        out_specs=[
            pl.BlockSpec((1, 1, C), lambda i: (i, 0, 0)),
            pl.BlockSpec((1, 1, C), lambda i: (i, 0, 0)),
        ],
        out_shape=[
            jax.ShapeDtypeStruct((nb, 1, C), _F32),
            jax.ShapeDtypeStruct((nb, 1, C), _F32),
        ],
    )(P, QgR)


def _bnmm_max(P, QgR, a, sh, wb, bb, Cp):
    n, C = P.shape
    C2 = wb.shape[1]
    blk = 128

    def body(p_ref, qg_ref, a_ref, sh_ref, wb_ref, bb_ref, o_ref):
        p = p_ref[...]
        av = a_ref[...]
        sv = sh_ref[...]
        w = wb_ref[...]
        acc = None
        for k in range(_K):
            z = p + qg_ref[:, k * Cp : k * Cp + C]
            h = jnp.maximum(z * av + sv, 0.0)
            y = jnp.dot(h, w, preferred_element_type=_F32)
            acc = y if acc is None else jnp.maximum(acc, y)
        o_ref[...] = acc + bb_ref[...]

    return pl.pallas_call(
        body,
        grid=(n // blk,),
        in_specs=[
            pl.BlockSpec((blk, C), lambda i: (i, 0)),
            pl.BlockSpec((blk, _K * Cp), lambda i: (i, 0)),
            pl.BlockSpec((1, C), lambda i: (0, 0)),
            pl.BlockSpec((1, C), lambda i: (0, 0)),
            pl.BlockSpec((C, C2), lambda i: (0, 0)),
            pl.BlockSpec((1, C2), lambda i: (0, 0)),
        ],
        out_specs=pl.BlockSpec((blk, C2), lambda i: (i, 0)),
        out_shape=jax.ShapeDtypeStruct((n, C2), _F32),
    )(P, QgR, a, sh, wb, bb)


def _head(x1, x2, x3, w4, b4, w5, b5, w6, b6, w7, b7):
    n = x1.shape[0]
    blk = 128
    w4a, w4b, w4c = w4[:64], w4[64:192], w4[192:]

    def body(x1r, x2r, x3r, w4ar, w4br, w4cr, b4r, w5r, b5r, w6r, b6r, w7r, b7r, o_ref):
        h = (
            jnp.dot(x1r[...], w4ar[...], preferred_element_type=_F32)
            + jnp.dot(x2r[...], w4br[...], preferred_element_type=_F32)
            + jnp.dot(x3r[...], w4cr[...], preferred_element_type=_F32)
            + b4r[...]
        )
        h = jnp.maximum(h, 0.0)
        h = jnp.maximum(jnp.dot(h, w5r[...], preferred_element_type=_F32) + b5r[...], 0.0)
        h = jnp.maximum(jnp.dot(h, w6r[...], preferred_element_type=_F32) + b6r[...], 0.0)
        o = jnp.dot(h, w7r[...], preferred_element_type=_F32) + b7r[...]
        m = jnp.max(o, axis=1, keepdims=True)
        shifted = o - m
        lse = jnp.log(jnp.sum(jnp.exp(shifted), axis=1, keepdims=True))
        o_ref[...] = shifted - lse

    full = lambda r, c: pl.BlockSpec((r, c), lambda i: (0, 0))
    return pl.pallas_call(
        body,
        grid=(n // blk,),
        in_specs=[
            pl.BlockSpec((blk, 64), lambda i: (i, 0)),
            pl.BlockSpec((blk, 128), lambda i: (i, 0)),
            pl.BlockSpec((blk, 256), lambda i: (i, 0)),
            full(64, 1024),
            full(128, 1024),
            full(256, 1024),
            full(1, 1024),
            full(1024, 512),
            full(1, 512),
            full(512, 256),
            full(1, 256),
            full(256, 10),
            full(1, 10),
        ],
        out_specs=pl.BlockSpec((blk, 10), lambda i: (i, 0)),
        out_shape=jax.ShapeDtypeStruct((n, 10), _F32),
    )(
        x1, x2, x3, w4a, w4b, w4c,
        b4.reshape(1, -1), w5, b5.reshape(1, -1),
        w6, b6.reshape(1, -1), w7, b7.reshape(1, -1),
    )


def _edge_layer(feat, featT, batch_col, batch_row, wa, ba, g, bt, wb, bb):
    d = feat.shape[1]
    C = wa.shape[1]
    Cp = max(C, 128)
    wi = wa[:d] - wa[d:]
    wj = wa[d:]
    if Cp != C:
        wj = jnp.pad(wj, ((0, 0), (0, Cp - C)))
    P, Q = _linear_pq(feat, wi, wj, ba.reshape(1, C))
    idx = _dist_topk(feat, featT, batch_col, batch_row)
    Qg = _sc_gather(Q, idx.reshape(-1))
    QgR = Qg.reshape(_N, _K * Cp)
    s1p, s2p = _edge_stats(P, QgR, Cp)
    cnt = float(_N * _K)
    s1 = jnp.sum(s1p.reshape(-1, C), axis=0)
    s2 = jnp.sum(s2p.reshape(-1, C), axis=0)
    mu = s1 / cnt
    var = s2 / cnt - mu * mu
    a = g / jnp.sqrt(var + 1e-5)
    sh = bt - mu * a
    return _bnmm_max(P, QgR, a.reshape(1, C), sh.reshape(1, C), wb, bb.reshape(1, C), Cp)


def kernel(x, batch, w1a, b1a, g1a, bt1a, w1b, b1b, w2a, b2a, g2a, bt2a, w2b, b2b,
           w3a, b3a, g3a, bt3a, w3b, b3b, w4, b4, w5, b5, w6, b6, w7, b7):
    batch = batch.astype(jnp.int32)
    bc = batch.reshape(_N, 1)
    br = batch.reshape(1, _N)
    x1 = _edge_layer(x, x.reshape(1, _N), bc, br, w1a, b1a, g1a, bt1a, w1b, b1b)
    x2 = _edge_layer(x1, x1.T, bc, br, w2a, b2a, g2a, bt2a, w2b, b2b)
    x3 = _edge_layer(x2, x2.T, bc, br, w3a, b3a, g3a, bt3a, w3b, b3b)
    return _head(x1, x2, x3, w4, b4, w5, b5, w6, b6, w7, b7)


# speedup vs baseline: 12.1166x; 1.5773x over previous
import functools

import jax
import jax.numpy as jnp
from jax import lax
from jax.experimental import pallas as pl
from jax.experimental.pallas import tpu as pltpu
from jax.experimental.pallas import tpu_sc as plsc

_N = 4096
_K = 20
_F32 = jnp.float32


_CW = 256
_KP = 24


def _dist_topk_pq(feat, featT, batch_col, batch_row, lo_c, nch_c, wi, wj, ba):
    n, d = feat.shape
    C = wi.shape[-1]
    Cq = wj.shape[-1]
    blk = 128

    def body(lo_ref, nch_ref, f_ref, ft_ref, b_ref, bt_ref, wi_ref, wj_ref,
             ba_ref, idx_ref, p_ref, q_ref, dist_s, ids_s):
        inf = float("inf")
        i = pl.program_id(0)
        lo = lo_ref[i]
        nc = nch_ref[i]
        fb = f_ref[pl.ds(i * blk, blk), :]
        if d == 1:
            p_ref[...] = fb * wi_ref[...] + ba_ref[...]
            q_ref[...] = fb * wj_ref[...]
        else:
            p_ref[...] = (
                jnp.dot(fb, wi_ref[...], preferred_element_type=_F32) + ba_ref[...]
            )
            q_ref[...] = jnp.dot(fb, wj_ref[...], preferred_element_type=_F32)
        ftb = ft_ref[...]
        d2b = jnp.sum(ftb * ftb, axis=0, keepdims=True)
        btb = bt_ref[...]

        def build(c, _):
            s = lo + c * _CW
            fc = f_ref[pl.ds(s, _CW), :]
            d2c = jnp.sum(fc * fc, axis=1, keepdims=True)
            if d == 1:
                cross = fc * ftb
            else:
                cross = jnp.dot(fc, ftb, preferred_element_type=_F32)
            dd = d2c + d2b - 2.0 * cross
            bc = b_ref[pl.ds(s, _CW), :]
            dd = jnp.where(bc != btb, 1e9, dd)
            ids = (s + lax.broadcasted_iota(jnp.int32, (_CW, blk), 0)).astype(_F32)
            dist_s[pl.ds(c * _CW, _CW), :] = dd
            ids_s[pl.ds(c * _CW, _CW), :] = ids
            return 0

        lax.fori_loop(0, nc, build, 0)

        picks = []
        jprev = None
        for _ in range(_K):
            jp = jprev

            def pass_a(c, macc):
                dd = dist_s[pl.ds(c * _CW, _CW), :]
                if jp is not None:
                    ii = ids_s[pl.ds(c * _CW, _CW), :]
                    dd = jnp.where(ii == jp, inf, dd)
                    dist_s[pl.ds(c * _CW, _CW), :] = dd
                return jnp.minimum(macc, dd)

            macc = lax.fori_loop(0, nc, pass_a, jnp.full((_CW, blk), inf, _F32))
            m = jnp.min(macc, axis=0, keepdims=True)

            def pass_b(c, jacc):
                dd = dist_s[pl.ds(c * _CW, _CW), :]
                ii = ids_s[pl.ds(c * _CW, _CW), :]
                return jnp.minimum(jacc, jnp.where(dd == m, ii, inf))

            jacc = lax.fori_loop(0, nc, pass_b, jnp.full((_CW, blk), inf, _F32))
            j = jnp.min(jacc, axis=0, keepdims=True)
            picks.append(j)
            jprev = j
        picks += [picks[-1]] * (_KP - _K)
        idx_ref[...] = jnp.concatenate(picks, axis=0).astype(jnp.int32)

    grid_spec = pltpu.PrefetchScalarGridSpec(
        num_scalar_prefetch=2,
        grid=(n // blk,),
        in_specs=[
            pl.BlockSpec((n, d), lambda i, lo, nc: (0, 0)),
            pl.BlockSpec((d, blk), lambda i, lo, nc: (0, i)),
            pl.BlockSpec((n, 1), lambda i, lo, nc: (0, 0)),
            pl.BlockSpec((1, blk), lambda i, lo, nc: (0, i)),
            pl.BlockSpec((wi.shape[0], C), lambda i, lo, nc: (0, 0)),
            pl.BlockSpec((wj.shape[0], Cq), lambda i, lo, nc: (0, 0)),
            pl.BlockSpec((1, C), lambda i, lo, nc: (0, 0)),
        ],
        out_specs=[
            pl.BlockSpec((_KP, blk), lambda i, lo, nc: (0, i)),
            pl.BlockSpec((blk, C), lambda i, lo, nc: (i, 0)),
            pl.BlockSpec((blk, Cq), lambda i, lo, nc: (i, 0)),
        ],
        scratch_shapes=[
            pltpu.VMEM((n, blk), _F32),
            pltpu.VMEM((n, blk), _F32),
        ],
    )
    return pl.pallas_call(
        body,
        grid_spec=grid_spec,
        out_shape=[
            jax.ShapeDtypeStruct((_KP, n), jnp.int32),
            jax.ShapeDtypeStruct((n, C), _F32),
            jax.ShapeDtypeStruct((n, Cq), _F32),
        ],
    )(lo_c, nch_c, feat, featT, batch_col, batch_row, wi, wj, ba)


def _sc_gather(Q, idx_flat):
    B = idx_flat.shape[0]
    C = Q.shape[1]
    NW = 32
    CH = 128
    per_w = B // NW
    nch = per_w // CH
    mesh = plsc.VectorSubcoreMesh(core_axis_name="c", subcore_axis_name="s")

    @functools.partial(
        pl.kernel,
        mesh=mesh,
        out_type=jax.ShapeDtypeStruct((B, C), _F32),
        scratch_types=[
            pltpu.VMEM((CH,), jnp.int32),
            pltpu.VMEM((CH,), jnp.int32),
            pltpu.VMEM((CH, C), _F32),
            pltpu.VMEM((CH, C), _F32),
            pltpu.SemaphoreType.DMA,
            pltpu.SemaphoreType.DMA,
        ],
    )
    def gk(q_hbm, idx_hbm, out_hbm, idx_a, idx_b, rows_a, rows_b, sem_a, sem_b):
        wid = lax.axis_index("s") * 2 + lax.axis_index("c")
        base = wid * per_w
        idx_v = (idx_a, idx_b)
        rows_v = (rows_a, rows_b)
        sems = (sem_a, sem_b)
        pltpu.sync_copy(idx_hbm.at[pl.ds(base, CH)], idx_a)
        gat = pltpu.async_copy(q_hbm.at[idx_a], rows_a, sem_a)
        for c in range(nch):
            s = c % 2
            o = 1 - s
            if c + 1 < nch:
                off_n = base + (c + 1) * CH
                pltpu.sync_copy(idx_hbm.at[pl.ds(off_n, CH)], idx_v[o])
                nxt = pltpu.async_copy(q_hbm.at[idx_v[o]], rows_v[o], sems[o])
            gat.wait()
            pltpu.sync_copy(rows_v[s], out_hbm.at[pl.ds(base + c * CH, CH)])
            if c + 1 < nch:
                gat = nxt

    return gk(Q, idx_flat)


def _edge_stats(P, Qg3, Cp):
    n, C = P.shape
    blk = 128
    nb = n // blk

    def body(p_ref, qg_ref, s1_ref, s2_ref):
        p = p_ref[...]
        s1 = jnp.zeros((1, C), _F32)
        s2 = jnp.zeros((1, C), _F32)
        for k in range(_K):
            z = p + qg_ref[k, :, :C]
            s1 = s1 + jnp.sum(z, axis=0, keepdims=True)
            s2 = s2 + jnp.sum(z * z, axis=0, keepdims=True)
        s1_ref[...] = s1.reshape(1, 1, C)
        s2_ref[...] = s2.reshape(1, 1, C)

    return pl.pallas_call(
        body,
        grid=(nb,),
        in_specs=[
            pl.BlockSpec((blk, C), lambda i: (i, 0)),
            pl.BlockSpec((_K, blk, Cp), lambda i: (0, i, 0)),
        ],
        out_specs=[
            pl.BlockSpec((1, 1, C), lambda i: (i, 0, 0)),
            pl.BlockSpec((1, 1, C), lambda i: (i, 0, 0)),
        ],
        out_shape=[
            jax.ShapeDtypeStruct((nb, 1, C), _F32),
            jax.ShapeDtypeStruct((nb, 1, C), _F32),
        ],
    )(P, Qg3)


def _bnmm_max(P, Qg3, a, sh, wb, bb, Cp):
    n, C = P.shape
    C2 = wb.shape[1]
    blk = 128

    def body(p_ref, qg_ref, a_ref, sh_ref, wb_ref, bb_ref, o_ref):
        p = p_ref[...]
        av = a_ref[...]
        sv = sh_ref[...]
        w = wb_ref[...]
        acc = None
        for k in range(_K):
            z = p + qg_ref[k, :, :C]
            h = jnp.maximum(z * av + sv, 0.0)
            y = jnp.dot(h, w, preferred_element_type=_F32)
            acc = y if acc is None else jnp.maximum(acc, y)
        o_ref[...] = acc + bb_ref[...]

    return pl.pallas_call(
        body,
        grid=(n // blk,),
        in_specs=[
            pl.BlockSpec((blk, C), lambda i: (i, 0)),
            pl.BlockSpec((_K, blk, Cp), lambda i: (0, i, 0)),
            pl.BlockSpec((1, C), lambda i: (0, 0)),
            pl.BlockSpec((1, C), lambda i: (0, 0)),
            pl.BlockSpec((C, C2), lambda i: (0, 0)),
            pl.BlockSpec((1, C2), lambda i: (0, 0)),
        ],
        out_specs=pl.BlockSpec((blk, C2), lambda i: (i, 0)),
        out_shape=jax.ShapeDtypeStruct((n, C2), _F32),
    )(P, Qg3, a, sh, wb, bb)


def _head(x1, x2, x3, w4, b4, w5, b5, w6, b6, w7, b7):
    n = x1.shape[0]
    blk = 128
    w4a, w4b, w4c = w4[:64], w4[64:192], w4[192:]

    def body(x1r, x2r, x3r, w4ar, w4br, w4cr, b4r, w5r, b5r, w6r, b6r, w7r, b7r, o_ref):
        h = (
            jnp.dot(x1r[...], w4ar[...], preferred_element_type=_F32)
            + jnp.dot(x2r[...], w4br[...], preferred_element_type=_F32)
            + jnp.dot(x3r[...], w4cr[...], preferred_element_type=_F32)
            + b4r[...]
        )
        h = jnp.maximum(h, 0.0)
        h = jnp.maximum(jnp.dot(h, w5r[...], preferred_element_type=_F32) + b5r[...], 0.0)
        h = jnp.maximum(jnp.dot(h, w6r[...], preferred_element_type=_F32) + b6r[...], 0.0)
        o = jnp.dot(h, w7r[...], preferred_element_type=_F32) + b7r[...]
        m = jnp.max(o, axis=1, keepdims=True)
        shifted = o - m
        lse = jnp.log(jnp.sum(jnp.exp(shifted), axis=1, keepdims=True))
        o_ref[...] = shifted - lse

    full = lambda r, c: pl.BlockSpec((r, c), lambda i: (0, 0))
    return pl.pallas_call(
        body,
        grid=(n // blk,),
        in_specs=[
            pl.BlockSpec((blk, 64), lambda i: (i, 0)),
            pl.BlockSpec((blk, 128), lambda i: (i, 0)),
            pl.BlockSpec((blk, 256), lambda i: (i, 0)),
            full(64, 1024),
            full(128, 1024),
            full(256, 1024),
            full(1, 1024),
            full(1024, 512),
            full(1, 512),
            full(512, 256),
            full(1, 256),
            full(256, 10),
            full(1, 10),
        ],
        out_specs=pl.BlockSpec((blk, 10), lambda i: (i, 0)),
        out_shape=jax.ShapeDtypeStruct((n, 10), _F32),
    )(
        x1, x2, x3, w4a, w4b, w4c,
        b4.reshape(1, -1), w5, b5.reshape(1, -1),
        w6, b6.reshape(1, -1), w7, b7.reshape(1, -1),
    )


def _edge_layer(feat, featT, batch_col, batch_row, lo_c, nch_c, wa, ba, g, bt, wb, bb):
    d = feat.shape[1]
    C = wa.shape[1]
    Cp = max(C, 128)
    wi = wa[:d] - wa[d:]
    wj = wa[d:]
    if Cp != C:
        wj = jnp.pad(wj, ((0, 0), (0, Cp - C)))
    idxT, P, Q = _dist_topk_pq(
        feat, featT, batch_col, batch_row, lo_c, nch_c, wi, wj, ba.reshape(1, C)
    )
    Qg = _sc_gather(Q, idxT[:_K].reshape(-1))
    Qg3 = Qg.reshape(_K, _N, Cp)
    s1p, s2p = _edge_stats(P, Qg3, Cp)
    cnt = float(_N * _K)
    s1 = jnp.sum(s1p.reshape(-1, C), axis=0)
    s2 = jnp.sum(s2p.reshape(-1, C), axis=0)
    mu = s1 / cnt
    var = s2 / cnt - mu * mu
    a = g / jnp.sqrt(var + 1e-5)
    sh = bt - mu * a
    return _bnmm_max(P, Qg3, a.reshape(1, C), sh.reshape(1, C), wb, bb.reshape(1, C), Cp)


def kernel(x, batch, w1a, b1a, g1a, bt1a, w1b, b1b, w2a, b2a, g2a, bt2a, w2b, b2b,
           w3a, b3a, g3a, bt3a, w3b, b3b, w4, b4, w5, b5, w6, b6, w7, b7):
    batch = batch.astype(jnp.int32)
    bc = batch.reshape(_N, 1)
    br = batch.reshape(1, _N)
    classes = jnp.arange(4, dtype=batch.dtype)
    starts = jnp.searchsorted(batch, classes, side="left").astype(jnp.int32)
    ends = jnp.searchsorted(batch, classes, side="right").astype(jnp.int32)
    bfirst = batch[:: 128]
    blast = batch[127 :: 128]
    lo_c = (starts[bfirst] // _CW) * _CW
    nch_c = (ends[blast] - lo_c + _CW - 1) // _CW
    x1 = _edge_layer(x, x.reshape(1, _N), bc, br, lo_c, nch_c,
                     w1a, b1a, g1a, bt1a, w1b, b1b)
    x2 = _edge_layer(x1, x1.T, bc, br, lo_c, nch_c, w2a, b2a, g2a, bt2a, w2b, b2b)
    x3 = _edge_layer(x2, x2.T, bc, br, lo_c, nch_c, w3a, b3a, g3a, bt3a, w3b, b3b)
    return _head(x1, x2, x3, w4, b4, w5, b5, w6, b6, w7, b7)


# speedup vs baseline: 12.6639x; 1.0452x over previous
import functools

import jax
import jax.numpy as jnp
from jax import lax
from jax.experimental import pallas as pl
from jax.experimental.pallas import tpu as pltpu
from jax.experimental.pallas import tpu_sc as plsc

_N = 4096
_K = 20
_F32 = jnp.float32


_CW = 512
_KP = 24


def _dist_topk_pq(feat, featT, batch_col, batch_row, lo_c, nch_c, wi, wj, ba):
    n, d = feat.shape
    C = wi.shape[-1]
    Cq = wj.shape[-1]
    blk = 128

    def body(lo_ref, nch_ref, f_ref, ft_ref, b_ref, bt_ref, wi_ref, wj_ref,
             ba_ref, idx_ref, p_ref, q_ref, dist_s, ids_s):
        inf = float("inf")
        i = pl.program_id(0)
        lo = lo_ref[i]
        nc = nch_ref[i]
        fb = f_ref[pl.ds(i * blk, blk), :]
        if d == 1:
            p_ref[...] = fb * wi_ref[...] + ba_ref[...]
            q_ref[...] = fb * wj_ref[...]
        else:
            p_ref[...] = (
                jnp.dot(fb, wi_ref[...], preferred_element_type=_F32) + ba_ref[...]
            )
            q_ref[...] = jnp.dot(fb, wj_ref[...], preferred_element_type=_F32)
        ftb = ft_ref[...]
        d2b = jnp.sum(ftb * ftb, axis=0, keepdims=True)
        btb = bt_ref[...]

        def build(c, _):
            s = lo + c * _CW
            fc = f_ref[pl.ds(s, _CW), :]
            d2c = jnp.sum(fc * fc, axis=1, keepdims=True)
            if d == 1:
                cross = fc * ftb
            else:
                cross = jnp.dot(fc, ftb, preferred_element_type=_F32)
            dd = d2c + d2b - 2.0 * cross
            bc = b_ref[pl.ds(s, _CW), :]
            dd = jnp.where(bc != btb, 1e9, dd)
            ids = (s + lax.broadcasted_iota(jnp.int32, (_CW, blk), 0)).astype(_F32)
            dist_s[pl.ds(c * _CW, _CW), :] = dd
            ids_s[pl.ds(c * _CW, _CW), :] = ids
            return 0

        lax.fori_loop(0, nc, build, 0)

        picks = []
        jprev = None
        for _ in range(_K):
            jp = jprev

            def fpass(c, carry):
                m, j = carry
                dd = dist_s[pl.ds(c * _CW, _CW), :]
                ii = ids_s[pl.ds(c * _CW, _CW), :]
                if jp is not None:
                    dd = jnp.where(ii == jp, inf, dd)
                    dist_s[pl.ds(c * _CW, _CW), :] = dd
                mc = jnp.min(dd, axis=0, keepdims=True)
                jc = jnp.min(jnp.where(dd == mc, ii, inf), axis=0, keepdims=True)
                jn = jnp.where(mc < m, jc, jnp.where(mc == m, jnp.minimum(j, jc), j))
                return (jnp.minimum(m, mc), jn)

            m, j = lax.fori_loop(
                0, nc, fpass,
                (jnp.full((1, blk), inf, _F32), jnp.full((1, blk), inf, _F32)),
            )
            picks.append(j)
            jprev = j
        picks += [picks[-1]] * (_KP - _K)
        idx_ref[...] = jnp.concatenate(picks, axis=0).astype(jnp.int32)

    grid_spec = pltpu.PrefetchScalarGridSpec(
        num_scalar_prefetch=2,
        grid=(n // blk,),
        in_specs=[
            pl.BlockSpec((n, d), lambda i, lo, nc: (0, 0)),
            pl.BlockSpec((d, blk), lambda i, lo, nc: (0, i)),
            pl.BlockSpec((n, 1), lambda i, lo, nc: (0, 0)),
            pl.BlockSpec((1, blk), lambda i, lo, nc: (0, i)),
            pl.BlockSpec((wi.shape[0], C), lambda i, lo, nc: (0, 0)),
            pl.BlockSpec((wj.shape[0], Cq), lambda i, lo, nc: (0, 0)),
            pl.BlockSpec((1, C), lambda i, lo, nc: (0, 0)),
        ],
        out_specs=[
            pl.BlockSpec((_KP, blk), lambda i, lo, nc: (0, i)),
            pl.BlockSpec((blk, C), lambda i, lo, nc: (i, 0)),
            pl.BlockSpec((blk, Cq), lambda i, lo, nc: (i, 0)),
        ],
        scratch_shapes=[
            pltpu.VMEM((n, blk), _F32),
            pltpu.VMEM((n, blk), _F32),
        ],
    )
    return pl.pallas_call(
        body,
        grid_spec=grid_spec,
        out_shape=[
            jax.ShapeDtypeStruct((_KP, n), jnp.int32),
            jax.ShapeDtypeStruct((n, C), _F32),
            jax.ShapeDtypeStruct((n, Cq), _F32),
        ],
    )(lo_c, nch_c, feat, featT, batch_col, batch_row, wi, wj, ba)


def _sc_gather(Q, idx_flat):
    B = idx_flat.shape[0]
    C = Q.shape[1]
    NW = 32
    CH = 128
    per_w = B // NW
    nch = per_w // CH
    mesh = plsc.VectorSubcoreMesh(core_axis_name="c", subcore_axis_name="s")

    @functools.partial(
        pl.kernel,
        mesh=mesh,
        out_type=jax.ShapeDtypeStruct((B, C), _F32),
        scratch_types=[
            pltpu.VMEM((CH,), jnp.int32),
            pltpu.VMEM((CH,), jnp.int32),
            pltpu.VMEM((CH, C), _F32),
            pltpu.VMEM((CH, C), _F32),
            pltpu.SemaphoreType.DMA,
            pltpu.SemaphoreType.DMA,
        ],
    )
    def gk(q_hbm, idx_hbm, out_hbm, idx_a, idx_b, rows_a, rows_b, sem_a, sem_b):
        wid = lax.axis_index("s") * 2 + lax.axis_index("c")
        base = wid * per_w
        idx_v = (idx_a, idx_b)
        rows_v = (rows_a, rows_b)
        sems = (sem_a, sem_b)
        pltpu.sync_copy(idx_hbm.at[pl.ds(base, CH)], idx_a)
        gat = pltpu.async_copy(q_hbm.at[idx_a], rows_a, sem_a)
        for c in range(nch):
            s = c % 2
            o = 1 - s
            if c + 1 < nch:
                off_n = base + (c + 1) * CH
                pltpu.sync_copy(idx_hbm.at[pl.ds(off_n, CH)], idx_v[o])
                nxt = pltpu.async_copy(q_hbm.at[idx_v[o]], rows_v[o], sems[o])
            gat.wait()
            pltpu.sync_copy(rows_v[s], out_hbm.at[pl.ds(base + c * CH, CH)])
            if c + 1 < nch:
                gat = nxt

    return gk(Q, idx_flat)


def _edge_stats(P, Qg3, Cp):
    n, C = P.shape
    blk = 128
    nb = n // blk

    def body(p_ref, qg_ref, s1_ref, s2_ref):
        p = p_ref[...]
        s1 = jnp.zeros((1, C), _F32)
        s2 = jnp.zeros((1, C), _F32)
        for k in range(_K):
            z = p + qg_ref[k, :, :C]
            s1 = s1 + jnp.sum(z, axis=0, keepdims=True)
            s2 = s2 + jnp.sum(z * z, axis=0, keepdims=True)
        s1_ref[...] = s1.reshape(1, 1, C)
        s2_ref[...] = s2.reshape(1, 1, C)

    return pl.pallas_call(
        body,
        grid=(nb,),
        in_specs=[
            pl.BlockSpec((blk, C), lambda i: (i, 0)),
            pl.BlockSpec((_K, blk, Cp), lambda i: (0, i, 0)),
        ],
        out_specs=[
            pl.BlockSpec((1, 1, C), lambda i: (i, 0, 0)),
            pl.BlockSpec((1, 1, C), lambda i: (i, 0, 0)),
        ],
        out_shape=[
            jax.ShapeDtypeStruct((nb, 1, C), _F32),
            jax.ShapeDtypeStruct((nb, 1, C), _F32),
        ],
    )(P, Qg3)


def _bnmm_max(P, Qg3, a, sh, wb, bb, Cp):
    n, C = P.shape
    C2 = wb.shape[1]
    blk = 128

    def body(p_ref, qg_ref, a_ref, sh_ref, wb_ref, bb_ref, o_ref):
        p = p_ref[...]
        av = a_ref[...]
        sv = sh_ref[...]
        w = wb_ref[...]
        acc = None
        for k in range(_K):
            z = p + qg_ref[k, :, :C]
            h = jnp.maximum(z * av + sv, 0.0)
            y = jnp.dot(h, w, preferred_element_type=_F32)
            acc = y if acc is None else jnp.maximum(acc, y)
        o_ref[...] = acc + bb_ref[...]

    return pl.pallas_call(
        body,
        grid=(n // blk,),
        in_specs=[
            pl.BlockSpec((blk, C), lambda i: (i, 0)),
            pl.BlockSpec((_K, blk, Cp), lambda i: (0, i, 0)),
            pl.BlockSpec((1, C), lambda i: (0, 0)),
            pl.BlockSpec((1, C), lambda i: (0, 0)),
            pl.BlockSpec((C, C2), lambda i: (0, 0)),
            pl.BlockSpec((1, C2), lambda i: (0, 0)),
        ],
        out_specs=pl.BlockSpec((blk, C2), lambda i: (i, 0)),
        out_shape=jax.ShapeDtypeStruct((n, C2), _F32),
    )(P, Qg3, a, sh, wb, bb)


def _head(x1, x2, x3, w4, b4, w5, b5, w6, b6, w7, b7):
    n = x1.shape[0]
    blk = 128
    w4a, w4b, w4c = w4[:64], w4[64:192], w4[192:]

    def body(x1r, x2r, x3r, w4ar, w4br, w4cr, b4r, w5r, b5r, w6r, b6r, w7r, b7r, o_ref):
        h = (
            jnp.dot(x1r[...], w4ar[...], preferred_element_type=_F32)
            + jnp.dot(x2r[...], w4br[...], preferred_element_type=_F32)
            + jnp.dot(x3r[...], w4cr[...], preferred_element_type=_F32)
            + b4r[...]
        )
        h = jnp.maximum(h, 0.0)
        h = jnp.maximum(jnp.dot(h, w5r[...], preferred_element_type=_F32) + b5r[...], 0.0)
        h = jnp.maximum(jnp.dot(h, w6r[...], preferred_element_type=_F32) + b6r[...], 0.0)
        o = jnp.dot(h, w7r[...], preferred_element_type=_F32) + b7r[...]
        m = jnp.max(o, axis=1, keepdims=True)
        shifted = o - m
        lse = jnp.log(jnp.sum(jnp.exp(shifted), axis=1, keepdims=True))
        o_ref[...] = shifted - lse

    full = lambda r, c: pl.BlockSpec((r, c), lambda i: (0, 0))
    return pl.pallas_call(
        body,
        grid=(n // blk,),
        in_specs=[
            pl.BlockSpec((blk, 64), lambda i: (i, 0)),
            pl.BlockSpec((blk, 128), lambda i: (i, 0)),
            pl.BlockSpec((blk, 256), lambda i: (i, 0)),
            full(64, 1024),
            full(128, 1024),
            full(256, 1024),
            full(1, 1024),
            full(1024, 512),
            full(1, 512),
            full(512, 256),
            full(1, 256),
            full(256, 10),
            full(1, 10),
        ],
        out_specs=pl.BlockSpec((blk, 10), lambda i: (i, 0)),
        out_shape=jax.ShapeDtypeStruct((n, 10), _F32),
    )(
        x1, x2, x3, w4a, w4b, w4c,
        b4.reshape(1, -1), w5, b5.reshape(1, -1),
        w6, b6.reshape(1, -1), w7, b7.reshape(1, -1),
    )


def _edge_layer(feat, featT, batch_col, batch_row, lo_c, nch_c, wa, ba, g, bt, wb, bb):
    d = feat.shape[1]
    C = wa.shape[1]
    Cp = max(C, 128)
    wi = wa[:d] - wa[d:]
    wj = wa[d:]
    if Cp != C:
        wj = jnp.pad(wj, ((0, 0), (0, Cp - C)))
    idxT, P, Q = _dist_topk_pq(
        feat, featT, batch_col, batch_row, lo_c, nch_c, wi, wj, ba.reshape(1, C)
    )
    Qg = _sc_gather(Q, idxT[:_K].reshape(-1))
    Qg3 = Qg.reshape(_K, _N, Cp)
    s1p, s2p = _edge_stats(P, Qg3, Cp)
    cnt = float(_N * _K)
    s1 = jnp.sum(s1p.reshape(-1, C), axis=0)
    s2 = jnp.sum(s2p.reshape(-1, C), axis=0)
    mu = s1 / cnt
    var = s2 / cnt - mu * mu
    a = g / jnp.sqrt(var + 1e-5)
    sh = bt - mu * a
    return _bnmm_max(P, Qg3, a.reshape(1, C), sh.reshape(1, C), wb, bb.reshape(1, C), Cp)


def kernel(x, batch, w1a, b1a, g1a, bt1a, w1b, b1b, w2a, b2a, g2a, bt2a, w2b, b2b,
           w3a, b3a, g3a, bt3a, w3b, b3b, w4, b4, w5, b5, w6, b6, w7, b7):
    batch = batch.astype(jnp.int32)
    bc = batch.reshape(_N, 1)
    br = batch.reshape(1, _N)
    classes = jnp.arange(4, dtype=batch.dtype)
    starts = jnp.searchsorted(batch, classes, side="left").astype(jnp.int32)
    ends = jnp.searchsorted(batch, classes, side="right").astype(jnp.int32)
    bfirst = batch[:: 128]
    blast = batch[127 :: 128]
    lo_c = (starts[bfirst] // _CW) * _CW
    nch_c = (ends[blast] - lo_c + _CW - 1) // _CW
    x1 = _edge_layer(x, x.reshape(1, _N), bc, br, lo_c, nch_c,
                     w1a, b1a, g1a, bt1a, w1b, b1b)
    x2 = _edge_layer(x1, x1.T, bc, br, lo_c, nch_c, w2a, b2a, g2a, bt2a, w2b, b2b)
    x3 = _edge_layer(x2, x2.T, bc, br, lo_c, nch_c, w3a, b3a, g3a, bt3a, w3b, b3b)
    return _head(x1, x2, x3, w4, b4, w5, b5, w6, b6, w7, b7)


# speedup vs baseline: 13.4584x; 1.0627x over previous
import functools

import jax
import jax.numpy as jnp
from jax import lax
from jax.experimental import pallas as pl
from jax.experimental.pallas import tpu as pltpu
from jax.experimental.pallas import tpu_sc as plsc

_N = 4096
_K = 20
_F32 = jnp.float32


_CW = 256
_KP = 24
_BLK = 256


def _dist_topk_pq(feat, featT, batch_col, batch_row, lo_c, nch_c, wi, wj, ba):
    n, d = feat.shape
    C = wi.shape[-1]
    Cq = wj.shape[-1]
    blk = _BLK

    def body(lo_ref, nch_ref, f_ref, ft_ref, b_ref, bt_ref, wi_ref, wj_ref,
             ba_ref, idx_ref, p_ref, q_ref, dist_s, ids_s):
        inf = float("inf")
        i = pl.program_id(0)
        lo = lo_ref[i]
        nc = nch_ref[i]
        fb = f_ref[pl.ds(i * blk, blk), :]
        if d == 1:
            p_ref[...] = fb * wi_ref[...] + ba_ref[...]
            q_ref[...] = fb * wj_ref[...]
        else:
            p_ref[...] = (
                jnp.dot(fb, wi_ref[...], preferred_element_type=_F32) + ba_ref[...]
            )
            q_ref[...] = jnp.dot(fb, wj_ref[...], preferred_element_type=_F32)
        ftb = ft_ref[...]
        d2b = jnp.sum(ftb * ftb, axis=0, keepdims=True)
        btb = bt_ref[...]

        def build(c, _):
            s = lo + c * _CW
            fc = f_ref[pl.ds(s, _CW), :]
            d2c = jnp.sum(fc * fc, axis=1, keepdims=True)
            if d == 1:
                cross = fc * ftb
            else:
                cross = jnp.dot(fc, ftb, preferred_element_type=_F32)
            dd = d2c + d2b - 2.0 * cross
            bc = b_ref[pl.ds(s, _CW), :]
            dd = jnp.where(bc != btb, 1e9, dd)
            ids = (s + lax.broadcasted_iota(jnp.int32, (_CW, blk), 0)).astype(_F32)
            dist_s[pl.ds(c * _CW, _CW), :] = dd
            ids_s[pl.ds(c * _CW, _CW), :] = ids
            return 0

        lax.fori_loop(0, nc, build, 0)

        picks = []
        jprev = None
        for _ in range(_K):
            jp = jprev

            def fpass(c, carry):
                m, j = carry
                dd = dist_s[pl.ds(c * _CW, _CW), :]
                ii = ids_s[pl.ds(c * _CW, _CW), :]
                if jp is not None:
                    dd = jnp.where(ii == jp, inf, dd)
                    dist_s[pl.ds(c * _CW, _CW), :] = dd
                mc = jnp.min(dd, axis=0, keepdims=True)
                jc = jnp.min(jnp.where(dd == mc, ii, inf), axis=0, keepdims=True)
                jn = jnp.where(mc < m, jc, jnp.where(mc == m, jnp.minimum(j, jc), j))
                return (jnp.minimum(m, mc), jn)

            m, j = lax.fori_loop(
                0, nc, fpass,
                (jnp.full((1, blk), inf, _F32), jnp.full((1, blk), inf, _F32)),
            )
            picks.append(j)
            jprev = j
        picks += [picks[-1]] * (_KP - _K)
        idx_ref[...] = jnp.concatenate(picks, axis=0).astype(jnp.int32)

    grid_spec = pltpu.PrefetchScalarGridSpec(
        num_scalar_prefetch=2,
        grid=(n // blk,),
        in_specs=[
            pl.BlockSpec((n, d), lambda i, lo, nc: (0, 0)),
            pl.BlockSpec((d, blk), lambda i, lo, nc: (0, i)),
            pl.BlockSpec((n, 1), lambda i, lo, nc: (0, 0)),
            pl.BlockSpec((1, blk), lambda i, lo, nc: (0, i)),
            pl.BlockSpec((wi.shape[0], C), lambda i, lo, nc: (0, 0)),
            pl.BlockSpec((wj.shape[0], Cq), lambda i, lo, nc: (0, 0)),
            pl.BlockSpec((1, C), lambda i, lo, nc: (0, 0)),
        ],
        out_specs=[
            pl.BlockSpec((_KP, blk), lambda i, lo, nc: (0, i)),
            pl.BlockSpec((blk, C), lambda i, lo, nc: (i, 0)),
            pl.BlockSpec((blk, Cq), lambda i, lo, nc: (i, 0)),
        ],
        scratch_shapes=[
            pltpu.VMEM((n, blk), _F32),
            pltpu.VMEM((n, blk), _F32),
        ],
    )
    return pl.pallas_call(
        body,
        grid_spec=grid_spec,
        out_shape=[
            jax.ShapeDtypeStruct((_KP, n), jnp.int32),
            jax.ShapeDtypeStruct((n, C), _F32),
            jax.ShapeDtypeStruct((n, Cq), _F32),
        ],
    )(lo_c, nch_c, feat, featT, batch_col, batch_row, wi, wj, ba)


def _sc_gather(Q, idx_flat):
    B = idx_flat.shape[0]
    C = Q.shape[1]
    NW = 32
    CH = 128
    per_w = B // NW
    nch = per_w // CH
    mesh = plsc.VectorSubcoreMesh(core_axis_name="c", subcore_axis_name="s")

    @functools.partial(
        pl.kernel,
        mesh=mesh,
        out_type=jax.ShapeDtypeStruct((B, C), _F32),
        scratch_types=[
            pltpu.VMEM((CH,), jnp.int32),
            pltpu.VMEM((CH,), jnp.int32),
            pltpu.VMEM((CH, C), _F32),
            pltpu.VMEM((CH, C), _F32),
            pltpu.SemaphoreType.DMA,
            pltpu.SemaphoreType.DMA,
        ],
    )
    def gk(q_hbm, idx_hbm, out_hbm, idx_a, idx_b, rows_a, rows_b, sem_a, sem_b):
        wid = lax.axis_index("s") * 2 + lax.axis_index("c")
        base = wid * per_w
        idx_v = (idx_a, idx_b)
        rows_v = (rows_a, rows_b)
        sems = (sem_a, sem_b)
        pltpu.sync_copy(idx_hbm.at[pl.ds(base, CH)], idx_a)
        gat = pltpu.async_copy(q_hbm.at[idx_a], rows_a, sem_a)
        for c in range(nch):
            s = c % 2
            o = 1 - s
            if c + 1 < nch:
                off_n = base + (c + 1) * CH
                pltpu.sync_copy(idx_hbm.at[pl.ds(off_n, CH)], idx_v[o])
                nxt = pltpu.async_copy(q_hbm.at[idx_v[o]], rows_v[o], sems[o])
            gat.wait()
            pltpu.sync_copy(rows_v[s], out_hbm.at[pl.ds(base + c * CH, CH)])
            if c + 1 < nch:
                gat = nxt

    return gk(Q, idx_flat)


def _edge_stats(P, Qg3, Cp):
    n, C = P.shape
    blk = 128
    nb = n // blk

    def body(p_ref, qg_ref, s1_ref, s2_ref):
        p = p_ref[...]
        s1 = jnp.zeros((1, C), _F32)
        s2 = jnp.zeros((1, C), _F32)
        for k in range(_K):
            z = p + qg_ref[k, :, :C]
            s1 = s1 + jnp.sum(z, axis=0, keepdims=True)
            s2 = s2 + jnp.sum(z * z, axis=0, keepdims=True)
        s1_ref[...] = s1.reshape(1, 1, C)
        s2_ref[...] = s2.reshape(1, 1, C)

    return pl.pallas_call(
        body,
        grid=(nb,),
        in_specs=[
            pl.BlockSpec((blk, C), lambda i: (i, 0)),
            pl.BlockSpec((_K, blk, Cp), lambda i: (0, i, 0)),
        ],
        out_specs=[
            pl.BlockSpec((1, 1, C), lambda i: (i, 0, 0)),
            pl.BlockSpec((1, 1, C), lambda i: (i, 0, 0)),
        ],
        out_shape=[
            jax.ShapeDtypeStruct((nb, 1, C), _F32),
            jax.ShapeDtypeStruct((nb, 1, C), _F32),
        ],
    )(P, Qg3)


def _bnmm_max(P, Qg3, a, sh, wb, bb, Cp):
    n, C = P.shape
    C2 = wb.shape[1]
    blk = 128

    def body(p_ref, qg_ref, a_ref, sh_ref, wb_ref, bb_ref, o_ref):
        p = p_ref[...]
        av = a_ref[...]
        sv = sh_ref[...]
        w = wb_ref[...]
        acc = None
        for k in range(_K):
            z = p + qg_ref[k, :, :C]
            h = jnp.maximum(z * av + sv, 0.0)
            y = jnp.dot(h, w, preferred_element_type=_F32)
            acc = y if acc is None else jnp.maximum(acc, y)
        o_ref[...] = acc + bb_ref[...]

    return pl.pallas_call(
        body,
        grid=(n // blk,),
        in_specs=[
            pl.BlockSpec((blk, C), lambda i: (i, 0)),
            pl.BlockSpec((_K, blk, Cp), lambda i: (0, i, 0)),
            pl.BlockSpec((1, C), lambda i: (0, 0)),
            pl.BlockSpec((1, C), lambda i: (0, 0)),
            pl.BlockSpec((C, C2), lambda i: (0, 0)),
            pl.BlockSpec((1, C2), lambda i: (0, 0)),
        ],
        out_specs=pl.BlockSpec((blk, C2), lambda i: (i, 0)),
        out_shape=jax.ShapeDtypeStruct((n, C2), _F32),
    )(P, Qg3, a, sh, wb, bb)


def _head(x1, x2, x3, w4, b4, w5, b5, w6, b6, w7, b7):
    n = x1.shape[0]
    blk = 128
    w4a, w4b, w4c = w4[:64], w4[64:192], w4[192:]

    def body(x1r, x2r, x3r, w4ar, w4br, w4cr, b4r, w5r, b5r, w6r, b6r, w7r, b7r, o_ref):
        h = (
            jnp.dot(x1r[...], w4ar[...], preferred_element_type=_F32)
            + jnp.dot(x2r[...], w4br[...], preferred_element_type=_F32)
            + jnp.dot(x3r[...], w4cr[...], preferred_element_type=_F32)
            + b4r[...]
        )
        h = jnp.maximum(h, 0.0)
        h = jnp.maximum(jnp.dot(h, w5r[...], preferred_element_type=_F32) + b5r[...], 0.0)
        h = jnp.maximum(jnp.dot(h, w6r[...], preferred_element_type=_F32) + b6r[...], 0.0)
        o = jnp.dot(h, w7r[...], preferred_element_type=_F32) + b7r[...]
        m = jnp.max(o, axis=1, keepdims=True)
        shifted = o - m
        lse = jnp.log(jnp.sum(jnp.exp(shifted), axis=1, keepdims=True))
        o_ref[...] = shifted - lse

    full = lambda r, c: pl.BlockSpec((r, c), lambda i: (0, 0))
    return pl.pallas_call(
        body,
        grid=(n // blk,),
        in_specs=[
            pl.BlockSpec((blk, 64), lambda i: (i, 0)),
            pl.BlockSpec((blk, 128), lambda i: (i, 0)),
            pl.BlockSpec((blk, 256), lambda i: (i, 0)),
            full(64, 1024),
            full(128, 1024),
            full(256, 1024),
            full(1, 1024),
            full(1024, 512),
            full(1, 512),
            full(512, 256),
            full(1, 256),
            full(256, 10),
            full(1, 10),
        ],
        out_specs=pl.BlockSpec((blk, 10), lambda i: (i, 0)),
        out_shape=jax.ShapeDtypeStruct((n, 10), _F32),
    )(
        x1, x2, x3, w4a, w4b, w4c,
        b4.reshape(1, -1), w5, b5.reshape(1, -1),
        w6, b6.reshape(1, -1), w7, b7.reshape(1, -1),
    )


def _edge_layer(feat, featT, batch_col, batch_row, lo_c, nch_c, wa, ba, g, bt, wb, bb):
    d = feat.shape[1]
    C = wa.shape[1]
    Cp = max(C, 128)
    wi = wa[:d] - wa[d:]
    wj = wa[d:]
    if Cp != C:
        wj = jnp.pad(wj, ((0, 0), (0, Cp - C)))
    idxT, P, Q = _dist_topk_pq(
        feat, featT, batch_col, batch_row, lo_c, nch_c, wi, wj, ba.reshape(1, C)
    )
    Qg = _sc_gather(Q, idxT[:_K].reshape(-1))
    Qg3 = Qg.reshape(_K, _N, Cp)
    s1p, s2p = _edge_stats(P, Qg3, Cp)
    cnt = float(_N * _K)
    s1 = jnp.sum(s1p.reshape(-1, C), axis=0)
    s2 = jnp.sum(s2p.reshape(-1, C), axis=0)
    mu = s1 / cnt
    var = s2 / cnt - mu * mu
    a = g / jnp.sqrt(var + 1e-5)
    sh = bt - mu * a
    return _bnmm_max(P, Qg3, a.reshape(1, C), sh.reshape(1, C), wb, bb.reshape(1, C), Cp)


def kernel(x, batch, w1a, b1a, g1a, bt1a, w1b, b1b, w2a, b2a, g2a, bt2a, w2b, b2b,
           w3a, b3a, g3a, bt3a, w3b, b3b, w4, b4, w5, b5, w6, b6, w7, b7):
    batch = batch.astype(jnp.int32)
    bc = batch.reshape(_N, 1)
    br = batch.reshape(1, _N)
    classes = jnp.arange(4, dtype=batch.dtype)
    starts = jnp.searchsorted(batch, classes, side="left").astype(jnp.int32)
    ends = jnp.searchsorted(batch, classes, side="right").astype(jnp.int32)
    bfirst = batch[:: _BLK]
    blast = batch[_BLK - 1 :: _BLK]
    lo_c = (starts[bfirst] // _CW) * _CW
    nch_c = (ends[blast] - lo_c + _CW - 1) // _CW
    x1 = _edge_layer(x, x.reshape(1, _N), bc, br, lo_c, nch_c,
                     w1a, b1a, g1a, bt1a, w1b, b1b)
    x2 = _edge_layer(x1, x1.T, bc, br, lo_c, nch_c, w2a, b2a, g2a, bt2a, w2b, b2b)
    x3 = _edge_layer(x2, x2.T, bc, br, lo_c, nch_c, w3a, b3a, g3a, bt3a, w3b, b3b)
    return _head(x1, x2, x3, w4, b4, w5, b5, w6, b6, w7, b7)
